# Initial kernel scaffold; baseline (speedup 1.0000x reference)
import jax
import jax.numpy as jnp
from jax.experimental import pallas as pl


def kernel(x, edge_index, etypes, emb, W, b, wih, whh, bih, bhh, gate_w, gate_b):
    raise NotImplementedError("write your pallas kernel here")


# speedup vs baseline: 16.1218x; 16.1218x over previous
import jax
import jax.numpy as jnp
from jax import lax
from jax.experimental import pallas as pl
from jax.experimental.pallas import tpu as pltpu
from jax.experimental.pallas import tpu_sc as plsc

N = 10000
E = 320000
D = 128
NETYPES = 3
NSTEPS = 3

NC = 2
NS = 16
NW = NC * NS

_sc_mesh = plsc.VectorSubcoreMesh(core_axis_name="c", subcore_axis_name="s",
                                  num_cores=NC, num_subcores=NS)

EMB_CH = 80
EMB_NCH = 4
EMB_PW = EMB_CH * EMB_NCH
EMB_PAD = NW * EMB_PW


def _embed_body(idx_hbm, emb_hbm, out_hbm, idxv, rows, sem):
    cid = lax.axis_index("c")
    sid = lax.axis_index("s")
    wid = sid * NC + cid
    pltpu.sync_copy(idx_hbm.at[wid], idxv)

    def body(j, carry):
        pltpu.async_copy(emb_hbm.at[idxv.at[j]], rows, sem).wait()
        pltpu.sync_copy(rows, out_hbm.at[pl.ds(wid * EMB_PW + j * EMB_CH, EMB_CH)])
        return carry

    lax.fori_loop(0, EMB_NCH, body, 0)


def _embed(xp, emb):
    out = pl.kernel(
        _embed_body,
        out_type=jax.ShapeDtypeStruct((EMB_PAD, D), jnp.float32),
        mesh=_sc_mesh,
        scratch_types=[
            pltpu.VMEM((EMB_NCH, EMB_CH), jnp.int32),
            pltpu.VMEM((EMB_CH, D), jnp.float32),
            pltpu.SemaphoreType.DMA,
        ],
    )(xp, emb)
    return out[:N]


ECH = 128
ENCH = 79
E_PW = ECH * ENCH
E_PAD = NW * E_PW
NPAD = N + 112
RPS = NPAD // NS


def _edge_body(gidx_hbm, dst_hbm, htab_hbm, zrows_hbm, out_hbm,
               gbuf, dbuf, rows, acc, sem):
    cid = lax.axis_index("c")
    sid = lax.axis_index("s")
    wid = sid * NC + cid
    pltpu.sync_copy(zrows_hbm, acc.at[pl.ds(sid * RPS, RPS)])
    pltpu.sync_copy(gidx_hbm.at[wid], gbuf)
    pltpu.sync_copy(dst_hbm.at[wid], dbuf)
    plsc.subcore_barrier()

    def body(j, carry):
        pltpu.async_copy(htab_hbm.at[gbuf.at[j]], rows, sem).wait()
        pltpu.sync_copy(rows, acc.at[dbuf.at[j]], add=True)
        return carry

    lax.fori_loop(0, ENCH, body, 0)
    plsc.subcore_barrier()
    pltpu.sync_copy(acc.at[pl.ds(sid * RPS, RPS)],
                    out_hbm.at[cid, pl.ds(sid * RPS, RPS)])


def _edge(htab, gidx_r, dst_r, zrows):
    return pl.kernel(
        _edge_body,
        out_type=jax.ShapeDtypeStruct((NC, NPAD, D), jnp.float32),
        mesh=_sc_mesh,
        scratch_types=[
            pltpu.VMEM((ENCH, ECH), jnp.int32),
            pltpu.VMEM((ENCH, ECH), jnp.int32),
            pltpu.VMEM((ECH, D), jnp.float32),
            pltpu.VMEM_SHARED((NPAD, D), jnp.float32),
            pltpu.SemaphoreType.DMA,
        ],
    )(gidx_r, dst_r, htab, zrows)


BN = 400
NBLK = N // BN


def _transform_body(h_ref, w_ref, b_ref, out_ref):
    h = h_ref[...]
    w = w_ref[...]
    b = b_ref[...]
    for k in range(NETYPES):
        out_ref[k] = lax.dot_general(
            h, w[k], (((1,), (1,)), ((), ())),
            preferred_element_type=jnp.float32) + b[k][None, :]


def _transform(h, W, b):
    return pl.pallas_call(
        _transform_body,
        out_shape=jax.ShapeDtypeStruct((NETYPES, N, D), jnp.float32),
        grid=(NBLK,),
        in_specs=[
            pl.BlockSpec((BN, D), lambda i: (i, 0)),
            pl.BlockSpec((NETYPES, D, D), lambda i: (0, 0, 0)),
            pl.BlockSpec((NETYPES, D), lambda i: (0, 0)),
        ],
        out_specs=pl.BlockSpec((NETYPES, BN, D), lambda i: (0, i, 0)),
    )(h, W, b)


def _gru_body(p_ref, h_ref, wih_ref, whh_ref, bih_ref, bhh_ref, out_ref):
    a = p_ref[0] + p_ref[1]
    h = h_ref[...]
    gi = lax.dot_general(a, wih_ref[...], (((1,), (1,)), ((), ())),
                         preferred_element_type=jnp.float32) + bih_ref[...]
    gh = lax.dot_general(h, whh_ref[...], (((1,), (1,)), ((), ())),
                         preferred_element_type=jnp.float32) + bhh_ref[...]
    ir, iz, inn = gi[:, :D], gi[:, D:2 * D], gi[:, 2 * D:]
    hr, hz, hn = gh[:, :D], gh[:, D:2 * D], gh[:, 2 * D:]
    r = jax.nn.sigmoid(ir + hr)
    z = jax.nn.sigmoid(iz + hz)
    n = jnp.tanh(inn + r * hn)
    out_ref[...] = (1.0 - z) * n + z * h


def _gru(parts, h, wih, whh, bih2, bhh2):
    return pl.pallas_call(
        _gru_body,
        out_shape=jax.ShapeDtypeStruct((N, D), jnp.float32),
        grid=(NBLK,),
        in_specs=[
            pl.BlockSpec((NC, BN, D), lambda i: (0, i, 0)),
            pl.BlockSpec((BN, D), lambda i: (i, 0)),
            pl.BlockSpec((3 * D, D), lambda i: (0, 0)),
            pl.BlockSpec((3 * D, D), lambda i: (0, 0)),
            pl.BlockSpec((1, 3 * D), lambda i: (0, 0)),
            pl.BlockSpec((1, 3 * D), lambda i: (0, 0)),
        ],
        out_specs=pl.BlockSpec((BN, D), lambda i: (i, 0)),
    )(parts, h, wih, whh, bih2, bhh2)


def _pool_body(h_ref, gw_ref, gb_ref, out_ref):
    h = h_ref[...]
    g = lax.dot_general(gw_ref[...], h, (((1,), (1,)), ((), ())),
                        preferred_element_type=jnp.float32) + gb_ref[0, 0]
    m = jnp.max(g)
    e = jnp.exp(g - m)
    attn = e * (1.0 / jnp.sum(e))
    out_ref[...] = lax.dot_general(attn, h, (((1,), (0,)), ((), ())),
                                   preferred_element_type=jnp.float32)


def _pool(h, gate_w, gb2):
    return pl.pallas_call(
        _pool_body,
        out_shape=jax.ShapeDtypeStruct((1, D), jnp.float32),
    )(h, gate_w, gb2)


def kernel(x, edge_index, etypes, emb, W, b, wih, whh, bih, bhh, gate_w, gate_b):
    x = x.astype(jnp.int32)
    src = edge_index[0].astype(jnp.int32)
    dst = edge_index[1].astype(jnp.int32)
    et = etypes.astype(jnp.int32)

    gidx = et * N + src
    gidx_r = jnp.zeros((E_PAD,), jnp.int32).at[:E].set(gidx).reshape(NW, ENCH, ECH)
    dst_r = jnp.full((E_PAD,), N, jnp.int32).at[:E].set(dst).reshape(NW, ENCH, ECH)
    zrows = jnp.zeros((RPS, D), jnp.float32)
    xp = jnp.zeros((EMB_PAD,), jnp.int32).at[:N].set(x).reshape(NW, EMB_NCH, EMB_CH)

    bih2 = bih.reshape(1, 3 * D)
    bhh2 = bhh.reshape(1, 3 * D)
    gb2 = gate_b.reshape(1, 1)

    h = _embed(xp, emb)
    for _ in range(NSTEPS):
        htab = _transform(h, W, b).reshape(NETYPES * N, D)
        parts = _edge(htab, gidx_r, dst_r, zrows)
        h = _gru(parts, h, wih, whh, bih2, bhh2)
    return _pool(h, gate_w, gb2)
